# baseline (speedup 1.0000x reference)
import jax
import jax.numpy as jnp
from jax.experimental import pallas as pl
from jax.experimental.pallas import tpu as pltpu

_HORIZON = 10


def _ode_body(n_batches, n_first, tlen, hid):
    n_rest = (n_batches - 1) * n_first
    S = 1 + n_rest

    def body(x_ref, w1_ref, b1_ref, w2_ref, b2_ref, w3_ref, b3_ref,
             dts_ref, out_ref):
        w1 = w1_ref[:, :]
        b1 = b1_ref[:, :]
        w2 = w2_ref[:, :]
        b2 = b2_ref[:, :]
        w3 = w3_ref[:, :]
        b3 = b3_ref[:, :]

        def g(s):
            h = jnp.broadcast_to(w1, (hid, S)) * jnp.broadcast_to(s, (hid, S))
            h = jnp.tanh(h + jnp.broadcast_to(b1, (hid, S)))
            h = jnp.dot(w2, h, preferred_element_type=jnp.float32)
            h = jnp.tanh(h + jnp.broadcast_to(b2, (hid, S)))
            out = jnp.dot(w3, h, preferred_element_type=jnp.float32)
            return out + jnp.broadcast_to(b3, (1, S))

        x0col = x_ref[0, :, tlen - 1:tlen]
        mu0 = jnp.mean(x0col, axis=0, keepdims=True)
        rest_rows = [
            jnp.transpose(x_ref[b, :, tlen - 1:tlen], (1, 0))
            for b in range(1, n_batches)
        ]
        s = jnp.concatenate([mu0] + rest_rows, axis=1)

        srows = [s]
        for i in range(_HORIZON - 1):
            dt = dts_ref[i]
            k1 = g(s)
            k2 = g(s + dt * k1 / 3.0)
            k3 = g(s + dt * (k2 - k1 / 3.0))
            k4 = g(s + dt * (k1 - k2 + k3))
            s = s + dt * (k1 + 3.0 * (k2 + k3) + k4) / 8.0
            srows.append(s)

        traj = jnp.concatenate(srows, axis=0)
        offs = traj[:, 0:1]
        offs = jnp.transpose(offs, (1, 0))
        offs = offs - offs[0:1, 0:1]
        out_ref[0, :, :] = (jnp.broadcast_to(x0col, (n_first, _HORIZON)) +
                            jnp.broadcast_to(offs, (n_first, _HORIZON)))
        rest = jnp.transpose(traj[:, 1:], (1, 0))
        for b in range(1, n_batches):
            out_ref[b, :, :] = rest[(b - 1) * n_first:b * n_first, :]

    return body


def kernel(x, W1, b1, W2, b2, W3, b3):
    Bx, Nx, Tx = x.shape
    hid = W1.shape[1]

    ts = jnp.linspace(0.0, float(_HORIZON), _HORIZON)
    dts = ts[1:] - ts[:-1]

    return pl.pallas_call(
        _ode_body(Bx, Nx, Tx, hid),
        out_shape=jax.ShapeDtypeStruct((Bx, Nx, _HORIZON), jnp.float32),
        in_specs=[pl.BlockSpec(memory_space=pltpu.VMEM)] * 7 +
                 [pl.BlockSpec(memory_space=pltpu.SMEM)],
        out_specs=pl.BlockSpec(memory_space=pltpu.VMEM),
    )(x, W1.reshape(hid, 1), b1.reshape(hid, 1), W2.T,
      b2.reshape(hid, 1), W3.reshape(1, hid), b3.reshape(1, 1), dts)


# speedup vs baseline: 1.2056x; 1.2056x over previous
import jax
import jax.numpy as jnp
from jax.experimental import pallas as pl
from jax.experimental.pallas import tpu as pltpu

_HORIZON = 10
_NCHUNK = 4


def _ode_body(n_first, n_rest, hid):
    S = 1 + n_rest
    base = (S // (128 * _NCHUNK)) * 128
    widths = [base] * (_NCHUNK - 1) + [S - base * (_NCHUNK - 1)]
    offs_start = [sum(widths[:i]) for i in range(_NCHUNK)]

    def body(x0_ref, yrest_ref, w1_ref, b1_ref, w2_ref, b2_ref, w3_ref,
             b3_ref, dts_ref, out0_ref, out1_ref):
        w1 = w1_ref[:, :]
        b1 = b1_ref[:, :]
        w2 = w2_ref[:, :]
        b2 = b2_ref[:, :]
        w3 = w3_ref[:, :]
        b3 = b3_ref[:, :]

        def gs(us):
            hs = [jnp.broadcast_to(w1, (hid, u.shape[1])) *
                  jnp.broadcast_to(u, (hid, u.shape[1])) for u in us]
            hs = [h + jnp.broadcast_to(b1, h.shape) for h in hs]
            hs = [jnp.tanh(h) for h in hs]
            hs = [jnp.dot(w2, h, preferred_element_type=jnp.float32)
                  for h in hs]
            hs = [h + jnp.broadcast_to(b2, h.shape) for h in hs]
            hs = [jnp.tanh(h) for h in hs]
            ks = [jnp.sum(h * jnp.broadcast_to(w3, h.shape), axis=0,
                          keepdims=True) for h in hs]
            return [k + jnp.broadcast_to(b3, k.shape) for k in ks]

        x0 = x0_ref[:, :]
        mu0 = jnp.mean(x0, axis=1, keepdims=True)
        yr = yrest_ref[:, :]
        full = jnp.concatenate([mu0, yr], axis=1)
        ss = [full[:, o:o + w] for o, w in zip(offs_start, widths)]

        for o, w, s in zip(offs_start, widths, ss):
            out1_ref[0:1, o:o + w] = s
        mus = [ss[0][0:1, 0:1]]
        for i in range(_HORIZON - 1):
            dt = dts_ref[i]
            k1 = gs(ss)
            k2 = gs([s + dt * k / 3.0 for s, k in zip(ss, k1)])
            k3 = gs([s + dt * (kb - ka / 3.0)
                     for s, ka, kb in zip(ss, k1, k2)])
            k4 = gs([s + dt * (ka - kb + kc)
                     for s, ka, kb, kc in zip(ss, k1, k2, k3)])
            ss = [s + dt * (ka + 3.0 * (kb + kc) + kd) / 8.0
                  for s, ka, kb, kc, kd in zip(ss, k1, k2, k3, k4)]
            for o, w, s in zip(offs_start, widths, ss):
                out1_ref[i + 1:i + 2, o:o + w] = s
            mus.append(ss[0][0:1, 0:1])

        offs = jnp.concatenate(mus, axis=0) - mus[0]
        out0_ref[:, :] = (jnp.broadcast_to(offs, (_HORIZON, n_first)) +
                          jnp.broadcast_to(x0, (_HORIZON, n_first)))

    return body


def kernel(x, W1, b1, W2, b2, W3, b3):
    Bx, Nx, Tx = x.shape
    hid = W1.shape[1]
    n_rest = (Bx - 1) * Nx

    last = x[:, :, -1]
    x0row = last[0].reshape(1, Nx)
    yrest = last[1:].reshape(1, n_rest)
    ts = jnp.linspace(0.0, float(_HORIZON), _HORIZON)
    dts = ts[1:] - ts[:-1]

    out0, out1 = pl.pallas_call(
        _ode_body(Nx, n_rest, hid),
        out_shape=(
            jax.ShapeDtypeStruct((_HORIZON, Nx), jnp.float32),
            jax.ShapeDtypeStruct((_HORIZON, 1 + n_rest), jnp.float32),
        ),
        in_specs=[pl.BlockSpec(memory_space=pltpu.VMEM)] * 8 +
                 [pl.BlockSpec(memory_space=pltpu.SMEM)],
        out_specs=(pl.BlockSpec(memory_space=pltpu.VMEM),
                   pl.BlockSpec(memory_space=pltpu.VMEM)),
    )(x0row, yrest, W1.reshape(hid, 1), b1.reshape(hid, 1), W2.T,
      b2.reshape(hid, 1), W3.reshape(hid, 1), b3.reshape(1, 1), dts)

    rest = out1[:, 1:].T.reshape(Bx - 1, Nx, _HORIZON)
    return jnp.concatenate([out0.T.reshape(1, Nx, _HORIZON), rest], axis=0)
